# baseline (speedup 1.0000x reference)
import functools

import jax
import jax.numpy as jnp
from jax import lax
from jax.experimental import pallas as pl
from jax.experimental.pallas import tpu as pltpu
from jax.experimental.pallas import tpu_sc as plsc

_N = 10000
_D = 128
_E = 320000
_BLK = 1000

_NCORES = 2
_NSUB = 16
_NW = _NCORES * _NSUB
_CH = 128
_NP = 10240
_RPT = _NP // _NSUB
_NCH = 84
_NRGN = 88
_EPT = _NRGN * _CH
_EPW = _E // _NW
_EPAD = _NW * _EPT


def _sc_edge_body(x_hbm, src_hbm, dst_hbm, att_hbm, u_out, s_out,
                  rows0, rows1, srcc0, srcc1, srcc2, dstc0, dstc1, dstc2,
                  attc0, attc1, attc2, u_sh, s_sh,
                  lsem0, lsem1, lsem2, gsem0, gsem1, ssem0, ssem1):
    c = lax.axis_index("c")
    s = lax.axis_index("s")
    w = c * _NSUB + s
    zv = jnp.zeros((16,), jnp.float32)
    rows_b = (rows0, rows1)
    srcc_b = (srcc0, srcc1, srcc2)
    dstc_b = (dstc0, dstc1, dstc2)
    attc_b = (attc0, attc1, attc2)
    gsem_b = (gsem0, gsem1)
    ssem_b = (ssem0, ssem1)
    lsem_b = (lsem0, lsem1, lsem2)
    base = w * _EPT

    def issue_loads(i_chunk, b3):
        off = base + i_chunk * _CH
        pltpu.async_copy(src_hbm.at[pl.ds(off, _CH)], srcc_b[b3],
                         lsem_b[b3])
        pltpu.async_copy(dst_hbm.at[pl.ds(off, _CH)], dstc_b[b3],
                         lsem_b[b3])
        pltpu.async_copy(att_hbm.at[pl.ds(off, _CH)], attc_b[b3],
                         lsem_b[b3])

    def wait_loads(b3):
        pltpu.make_async_copy(src_hbm.at[pl.ds(0, _CH)], srcc_b[b3],
                              lsem_b[b3]).wait()
        pltpu.make_async_copy(dst_hbm.at[pl.ds(0, _CH)], dstc_b[b3],
                              lsem_b[b3]).wait()
        pltpu.make_async_copy(att_hbm.at[pl.ds(0, _CH)], attc_b[b3],
                              lsem_b[b3]).wait()

    def issue_gather(b3, b2):
        pltpu.async_copy(x_hbm.at[srcc_b[b3]], rows_b[b2], gsem_b[b2])

    def wait_gather(b2):
        pltpu.make_async_copy(x_hbm.at[pl.ds(0, _CH)], rows_b[b2],
                              gsem_b[b2]).wait()

    def issue_scatter(b3, b2):
        pltpu.sync_copy(rows_b[b2], u_sh.at[dstc_b[b3]], add=True)
        pltpu.sync_copy(attc_b[b3], s_sh.at[dstc_b[b3]], add=True)

    def wait_scatter(b2):
        pass

    def scale(b3, b2):
        rows_v = rows_b[b2]
        att_c = attc_b[b3]

        def scale_body(g, carry2):
            av = att_c[pl.ds(g * 16, 16)]
            for l in range(16):
                a = av[l]
                k = g * 16 + l
                for j in range(8):
                    sl = pl.ds(j * 16, 16)
                    rows_v[k, sl] = rows_v[k, sl] * a
            return carry2
        lax.fori_loop(0, _CH // 16, scale_body, 0)

    issue_loads(0, 0)
    issue_loads(1, 1)

    def zrow_body(i, carry):
        for j in range(8):
            rows0[i, pl.ds(j * 16, 16)] = zv
        return carry
    lax.fori_loop(0, _CH, zrow_body, 0)

    row0 = s * _RPT
    for t in range(_RPT // _CH):
        pltpu.sync_copy(rows0, u_sh.at[pl.ds(row0 + t * _CH, _CH)])
        pltpu.sync_copy(rows0.at[0], s_sh.at[pl.ds(row0 + t * _CH, _CH)])
    plsc.subcore_barrier()

    wait_loads(0)
    issue_gather(0, 0)

    def body(o, carry):
        for u in range(6):
            i = o * 6 + u
            b2 = u % 2
            b3 = u % 3
            b3p1 = (u + 1) % 3
            b3p2 = (u + 2) % 3
            if u == 0:
                @pl.when(o > 0)
                def _():
                    wait_scatter(1 - b2)
            else:
                wait_scatter(1 - b2)
            issue_loads(i + 2, b3p2)
            wait_loads(b3p1)
            issue_gather(b3p1, 1 - b2)
            wait_gather(b2)
            scale(b3, b2)
            issue_scatter(b3, b2)
        return carry
    lax.fori_loop(0, _NCH // 6, body, 0)
    wait_scatter(1)
    wait_gather(0)
    wait_loads((_NCH + 1) % 3)
    plsc.subcore_barrier()

    pltpu.sync_copy(u_sh.at[pl.ds(row0, _RPT)],
                    u_out.at[c, pl.ds(row0, _RPT)])
    pltpu.sync_copy(s_sh.at[pl.ds(row0, _RPT)],
                    s_out.at[c, pl.ds(row0, _RPT)])


def _sc_edge_pass(x, src, dst, att):
    mesh = plsc.VectorSubcoreMesh(core_axis_name="c", subcore_axis_name="s")
    fn = functools.partial(
        pl.kernel,
        mesh=mesh,
        out_type=[
            jax.ShapeDtypeStruct((_NCORES, _NP, _D), jnp.float32),
            jax.ShapeDtypeStruct((_NCORES, _NP), jnp.float32),
        ],
        scratch_types=[
            pltpu.VMEM((_CH, _D), jnp.float32),
            pltpu.VMEM((_CH, _D), jnp.float32),
            pltpu.VMEM((_CH,), jnp.int32),
            pltpu.VMEM((_CH,), jnp.int32),
            pltpu.VMEM((_CH,), jnp.int32),
            pltpu.VMEM((_CH,), jnp.int32),
            pltpu.VMEM((_CH,), jnp.int32),
            pltpu.VMEM((_CH,), jnp.int32),
            pltpu.VMEM((_CH,), jnp.float32),
            pltpu.VMEM((_CH,), jnp.float32),
            pltpu.VMEM((_CH,), jnp.float32),
            pltpu.VMEM_SHARED((_NP, _D), jnp.float32),
            pltpu.VMEM_SHARED((_NP,), jnp.float32),
            pltpu.SemaphoreType.DMA,
            pltpu.SemaphoreType.DMA,
            pltpu.SemaphoreType.DMA,
            pltpu.SemaphoreType.DMA,
            pltpu.SemaphoreType.DMA,
            pltpu.SemaphoreType.DMA,
            pltpu.SemaphoreType.DMA,
        ],
    )(_sc_edge_body)
    return fn(x, src, dst, att)


def _dense_body(num0_ref, num1_ref, den0_ref, den1_ref, x_ref, w_ref,
                b_ref, g_ref, be_ref, o_ref):
    num = num0_ref[...] + num1_ref[...]
    den = den0_ref[...] + den1_ref[...]
    x = x_ref[...]
    aggr = jnp.where(den > 0.0, num / jnp.where(den > 0.0, den, 1.0), 0.0)
    gelu = 0.5 * aggr * (1.0 + jax.lax.erf(aggr * 0.7071067811865476))
    h = gelu + x
    t = jax.lax.dot_general(h, w_ref[...], (((1,), (1,)), ((), ())),
                            preferred_element_type=jnp.float32)
    t = t + b_ref[...]
    mu = jnp.mean(t, axis=-1, keepdims=True)
    var = jnp.mean((t - mu) ** 2, axis=-1, keepdims=True)
    o_ref[...] = (t - mu) * jax.lax.rsqrt(var + 1e-5) * g_ref[...] + be_ref[...]


def _dense_layer(num0, num1, den0, den1, x, w, b, g, be):
    row_spec = pl.BlockSpec((_BLK, _D), lambda i: (i, 0))
    den_spec = pl.BlockSpec((_BLK, 1), lambda i: (i, 0))
    vec_spec = pl.BlockSpec((1, _D), lambda i: (0, 0))
    return pl.pallas_call(
        _dense_body,
        grid=(_N // _BLK,),
        in_specs=[row_spec, row_spec, den_spec, den_spec, row_spec,
                  pl.BlockSpec((_D, _D), lambda i: (0, 0)),
                  vec_spec, vec_spec, vec_spec],
        out_specs=row_spec,
        out_shape=jax.ShapeDtypeStruct((_N, _D), jnp.float32),
    )(num0, num1, den0, den1, x, w, b, g, be)


def _per_worker_pad(arr, dtype):
    a = arr.reshape(_NW, _EPW)
    padded = jnp.concatenate(
        [a, jnp.zeros((_NW, _EPT - _EPW), dtype)], axis=1)
    return padded.reshape(_EPAD)


def kernel(node_attr, edge_index, batch_idx, adv_atts, W0, b0, g0, be0,
           W1, b1, g1, be1):
    src = _per_worker_pad(edge_index[0], jnp.int32)
    dst = _per_worker_pad(edge_index[1], jnp.int32)
    att0 = _per_worker_pad(adv_atts[0], jnp.float32)
    att1 = _per_worker_pad(adv_atts[1], jnp.float32)

    x = node_attr
    for att, w, b, g, be in ((att0, W0, b0, g0, be0),
                             (att1, W1, b1, g1, be1)):
        u, sden = _sc_edge_pass(x, src, dst, att)
        x = _dense_layer(u[0, :_N], u[1, :_N],
                         sden[0, :_N].reshape(_N, 1),
                         sden[1, :_N].reshape(_N, 1),
                         x, w, b.reshape(1, _D), g.reshape(1, _D),
                         be.reshape(1, _D))
    return x


# speedup vs baseline: 1.8969x; 1.8969x over previous
import functools

import jax
import jax.numpy as jnp
from jax import lax
from jax.experimental import pallas as pl
from jax.experimental.pallas import tpu as pltpu
from jax.experimental.pallas import tpu_sc as plsc

_N = 10000
_D = 128
_E = 320000
_BLK = 1000

_NCORES = 2
_NSUB = 16
_NW = _NCORES * _NSUB
_CH = 128
_NP = 10240
_RPT = _NP // _NSUB
_NCH = 80
_NRGN = 88
_EPT = _NRGN * _CH
_EPW = _E // _NW
_EPAD = _NW * _EPT


def _sc_edge_body(x_hbm, src_hbm, dst_hbm, att_hbm, u_out, s_out,
                  rows0, rows1, srcc0, srcc1, dstc0, dstc1,
                  attc0, attc1, u_sh, s_sh,
                  lsem0, lsem1, gsem0, gsem1):
    c = lax.axis_index("c")
    s = lax.axis_index("s")
    w = c * _NSUB + s
    zv = jnp.zeros((16,), jnp.float32)
    rows_b = (rows0, rows1)
    srcc_b = (srcc0, srcc1)
    dstc_b = (dstc0, dstc1)
    attc_b = (attc0, attc1)
    gsem_b = (gsem0, gsem1)
    lsem_b = (lsem0, lsem1)
    base = w * _EPT

    def issue_loads(i_chunk, b3):
        off = base + i_chunk * _CH
        pltpu.async_copy(src_hbm.at[pl.ds(off, _CH)], srcc_b[b3],
                         lsem_b[b3])
        pltpu.async_copy(dst_hbm.at[pl.ds(off, _CH)], dstc_b[b3],
                         lsem_b[b3])
        pltpu.async_copy(att_hbm.at[pl.ds(off, _CH)], attc_b[b3],
                         lsem_b[b3])

    def wait_loads(b3):
        pltpu.make_async_copy(src_hbm.at[pl.ds(0, _CH)], srcc_b[b3],
                              lsem_b[b3]).wait()
        pltpu.make_async_copy(dst_hbm.at[pl.ds(0, _CH)], dstc_b[b3],
                              lsem_b[b3]).wait()
        pltpu.make_async_copy(att_hbm.at[pl.ds(0, _CH)], attc_b[b3],
                              lsem_b[b3]).wait()

    def issue_gather(b):
        pltpu.async_copy(x_hbm.at[srcc_b[b]], rows_b[b], gsem_b[b])

    def wait_gather(b):
        pltpu.make_async_copy(x_hbm.at[pl.ds(0, _CH)], rows_b[b],
                              gsem_b[b]).wait()

    def issue_scatter(b):
        pltpu.sync_copy(rows_b[b], u_sh.at[dstc_b[b]], add=True)
        pltpu.sync_copy(attc_b[b], s_sh.at[dstc_b[b]], add=True)

    def scale(b):
        rows_v = rows_b[b]
        att_c = attc_b[b]

        def scale_body(g, carry2):
            av = att_c[pl.ds(g * 16, 16)]
            for l in range(16):
                a = av[l]
                k = g * 16 + l
                for j in range(8):
                    sl = pl.ds(j * 16, 16)
                    rows_v[k, sl] = rows_v[k, sl] * a
            return carry2
        lax.fori_loop(0, _CH // 16, scale_body, 0)

    issue_loads(0, 0)
    issue_loads(1, 1)

    def zrow_body(i, carry):
        for j in range(8):
            rows0[i, pl.ds(j * 16, 16)] = zv
        return carry
    lax.fori_loop(0, _CH, zrow_body, 0)

    row0 = s * _RPT
    for t in range(_RPT // _CH):
        pltpu.sync_copy(rows0, u_sh.at[pl.ds(row0 + t * _CH, _CH)])
        pltpu.sync_copy(rows0.at[0], s_sh.at[pl.ds(row0 + t * _CH, _CH)])
    plsc.subcore_barrier()

    wait_loads(0)
    issue_gather(0)

    def body(o, carry):
        for b in range(2):
            i = o * 2 + b
            wait_loads(1 - b)
            issue_gather(1 - b)
            wait_gather(b)
            scale(b)
            issue_scatter(b)
            issue_loads(i + 2, b)
        return carry
    lax.fori_loop(0, _NCH // 2, body, 0)
    wait_gather(0)
    wait_loads(1)
    plsc.subcore_barrier()

    pltpu.sync_copy(u_sh.at[pl.ds(row0, _RPT)],
                    u_out.at[c, pl.ds(row0, _RPT)])
    pltpu.sync_copy(s_sh.at[pl.ds(row0, _RPT)],
                    s_out.at[c, pl.ds(row0, _RPT)])


def _sc_edge_pass(x, src, dst, att):
    mesh = plsc.VectorSubcoreMesh(core_axis_name="c", subcore_axis_name="s")
    fn = functools.partial(
        pl.kernel,
        mesh=mesh,
        out_type=[
            jax.ShapeDtypeStruct((_NCORES, _NP, _D), jnp.float32),
            jax.ShapeDtypeStruct((_NCORES, _NP), jnp.float32),
        ],
        scratch_types=[
            pltpu.VMEM((_CH, _D), jnp.float32),
            pltpu.VMEM((_CH, _D), jnp.float32),
            pltpu.VMEM((_CH,), jnp.int32),
            pltpu.VMEM((_CH,), jnp.int32),
            pltpu.VMEM((_CH,), jnp.int32),
            pltpu.VMEM((_CH,), jnp.int32),
            pltpu.VMEM((_CH,), jnp.float32),
            pltpu.VMEM((_CH,), jnp.float32),
            pltpu.VMEM_SHARED((_NP, _D), jnp.float32),
            pltpu.VMEM_SHARED((_NP,), jnp.float32),
            pltpu.SemaphoreType.DMA,
            pltpu.SemaphoreType.DMA,
            pltpu.SemaphoreType.DMA,
            pltpu.SemaphoreType.DMA,
        ],
    )(_sc_edge_body)
    return fn(x, src, dst, att)


def _dense_body(num0_ref, num1_ref, den0_ref, den1_ref, x_ref, w_ref,
                b_ref, g_ref, be_ref, o_ref):
    num = num0_ref[...] + num1_ref[...]
    den = den0_ref[...] + den1_ref[...]
    x = x_ref[...]
    aggr = jnp.where(den > 0.0, num / jnp.where(den > 0.0, den, 1.0), 0.0)
    gelu = 0.5 * aggr * (1.0 + jax.lax.erf(aggr * 0.7071067811865476))
    h = gelu + x
    t = jax.lax.dot_general(h, w_ref[...], (((1,), (1,)), ((), ())),
                            preferred_element_type=jnp.float32)
    t = t + b_ref[...]
    mu = jnp.mean(t, axis=-1, keepdims=True)
    var = jnp.mean((t - mu) ** 2, axis=-1, keepdims=True)
    o_ref[...] = (t - mu) * jax.lax.rsqrt(var + 1e-5) * g_ref[...] + be_ref[...]


def _dense_layer(num0, num1, den0, den1, x, w, b, g, be):
    row_spec = pl.BlockSpec((_BLK, _D), lambda i: (i, 0))
    den_spec = pl.BlockSpec((_BLK, 1), lambda i: (i, 0))
    vec_spec = pl.BlockSpec((1, _D), lambda i: (0, 0))
    return pl.pallas_call(
        _dense_body,
        grid=(_N // _BLK,),
        in_specs=[row_spec, row_spec, den_spec, den_spec, row_spec,
                  pl.BlockSpec((_D, _D), lambda i: (0, 0)),
                  vec_spec, vec_spec, vec_spec],
        out_specs=row_spec,
        out_shape=jax.ShapeDtypeStruct((_N, _D), jnp.float32),
    )(num0, num1, den0, den1, x, w, b, g, be)


def _per_worker_pad(arr, dtype):
    a = arr.reshape(_NW, _EPW)
    padded = jnp.concatenate(
        [a, jnp.zeros((_NW, _EPT - _EPW), dtype)], axis=1)
    return padded.reshape(_EPAD)


def kernel(node_attr, edge_index, batch_idx, adv_atts, W0, b0, g0, be0,
           W1, b1, g1, be1):
    src = _per_worker_pad(edge_index[0], jnp.int32)
    dst = _per_worker_pad(edge_index[1], jnp.int32)
    att0 = _per_worker_pad(adv_atts[0], jnp.float32)
    att1 = _per_worker_pad(adv_atts[1], jnp.float32)

    x = node_attr
    for att, w, b, g, be in ((att0, W0, b0, g0, be0),
                             (att1, W1, b1, g1, be1)):
        u, sden = _sc_edge_pass(x, src, dst, att)
        x = _dense_layer(u[0, :_N], u[1, :_N],
                         sden[0, :_N].reshape(_N, 1),
                         sden[1, :_N].reshape(_N, 1),
                         x, w, b.reshape(1, _D), g.reshape(1, _D),
                         be.reshape(1, _D))
    return x


# speedup vs baseline: 2.0849x; 1.0991x over previous
import functools

import jax
import jax.numpy as jnp
from jax import lax
from jax.experimental import pallas as pl
from jax.experimental.pallas import tpu as pltpu
from jax.experimental.pallas import tpu_sc as plsc

_N = 10000
_D = 128
_E = 320000
_BLK = 1000

_NCORES = 2
_NSUB = 16
_NW = _NCORES * _NSUB
_CH = 128
_NP = 10240
_RPT = _NP // _NSUB
_NCH = 80
_NRGN = 88
_EPT = _NRGN * _CH
_EPW = _E // _NW
_EPAD = _NW * _EPT


def _sc_edge_body(x_hbm, src_hbm, dst_hbm, att_hbm, u_out, s_out,
                  rows0, srcc0, dstc0, attc0, u_sh, s_sh,
                  lsem0, gsem0):
    c = lax.axis_index("c")
    s = lax.axis_index("s")
    w = c * _NSUB + s
    zv = jnp.zeros((16,), jnp.float32)
    base = w * _EPT

    def issue_loads(i_chunk):
        off = base + i_chunk * _CH
        pltpu.async_copy(src_hbm.at[pl.ds(off, _CH)], srcc0, lsem0)
        pltpu.async_copy(dst_hbm.at[pl.ds(off, _CH)], dstc0, lsem0)
        pltpu.async_copy(att_hbm.at[pl.ds(off, _CH)], attc0, lsem0)

    def wait_loads():
        pltpu.make_async_copy(src_hbm.at[pl.ds(0, _CH)], srcc0,
                              lsem0).wait()
        pltpu.make_async_copy(dst_hbm.at[pl.ds(0, _CH)], dstc0,
                              lsem0).wait()
        pltpu.make_async_copy(att_hbm.at[pl.ds(0, _CH)], attc0,
                              lsem0).wait()

    def issue_scatter():
        pltpu.sync_copy(rows0, u_sh.at[dstc0], add=True)
        pltpu.sync_copy(attc0, s_sh.at[dstc0], add=True)

    def scale():
        rows_v = rows0
        att_c = attc0

        def scale_body(g, carry2):
            av = att_c[pl.ds(g * 16, 16)]
            for l in range(16):
                a = av[l]
                k = g * 16 + l
                for j in range(8):
                    sl = pl.ds(j * 16, 16)
                    rows_v[k, sl] = rows_v[k, sl] * a
            return carry2
        lax.fori_loop(0, _CH // 16, scale_body, 0)

    issue_loads(0)

    def zrow_body(i, carry):
        for j in range(8):
            rows0[i, pl.ds(j * 16, 16)] = zv
        return carry
    lax.fori_loop(0, _CH, zrow_body, 0)

    row0 = s * _RPT
    for t in range(_RPT // _CH):
        pltpu.sync_copy(rows0, u_sh.at[pl.ds(row0 + t * _CH, _CH)])
        pltpu.sync_copy(rows0.at[0], s_sh.at[pl.ds(row0 + t * _CH, _CH)])
    plsc.subcore_barrier()

    def body(i, carry):
        wait_loads()
        pltpu.async_copy(x_hbm.at[srcc0], rows0, gsem0).wait()
        scale()
        issue_scatter()
        issue_loads(i + 1)
        return carry
    lax.fori_loop(0, _NCH, body, 0)
    wait_loads()
    plsc.subcore_barrier()

    pltpu.sync_copy(u_sh.at[pl.ds(row0, _RPT)],
                    u_out.at[c, pl.ds(row0, _RPT)])
    pltpu.sync_copy(s_sh.at[pl.ds(row0, _RPT)],
                    s_out.at[c, pl.ds(row0, _RPT)])


def _sc_edge_pass(x, src, dst, att):
    mesh = plsc.VectorSubcoreMesh(core_axis_name="c", subcore_axis_name="s")
    fn = functools.partial(
        pl.kernel,
        mesh=mesh,
        out_type=[
            jax.ShapeDtypeStruct((_NCORES, _NP, _D), jnp.float32),
            jax.ShapeDtypeStruct((_NCORES, _NP), jnp.float32),
        ],
        scratch_types=[
            pltpu.VMEM((_CH, _D), jnp.float32),
            pltpu.VMEM((_CH,), jnp.int32),
            pltpu.VMEM((_CH,), jnp.int32),
            pltpu.VMEM((_CH,), jnp.float32),
            pltpu.VMEM_SHARED((_NP, _D), jnp.float32),
            pltpu.VMEM_SHARED((_NP,), jnp.float32),
            pltpu.SemaphoreType.DMA,
            pltpu.SemaphoreType.DMA,
        ],
    )(_sc_edge_body)
    return fn(x, src, dst, att)


def _dense_body(num0_ref, num1_ref, den0_ref, den1_ref, x_ref, w_ref,
                b_ref, g_ref, be_ref, o_ref):
    num = num0_ref[...] + num1_ref[...]
    den = den0_ref[...] + den1_ref[...]
    x = x_ref[...]
    aggr = jnp.where(den > 0.0, num / jnp.where(den > 0.0, den, 1.0), 0.0)
    gelu = 0.5 * aggr * (1.0 + jax.lax.erf(aggr * 0.7071067811865476))
    h = gelu + x
    t = jax.lax.dot_general(h, w_ref[...], (((1,), (1,)), ((), ())),
                            preferred_element_type=jnp.float32)
    t = t + b_ref[...]
    mu = jnp.mean(t, axis=-1, keepdims=True)
    var = jnp.mean((t - mu) ** 2, axis=-1, keepdims=True)
    o_ref[...] = (t - mu) * jax.lax.rsqrt(var + 1e-5) * g_ref[...] + be_ref[...]


def _dense_layer(num0, num1, den0, den1, x, w, b, g, be):
    row_spec = pl.BlockSpec((_BLK, _D), lambda i: (i, 0))
    den_spec = pl.BlockSpec((_BLK, 1), lambda i: (i, 0))
    vec_spec = pl.BlockSpec((1, _D), lambda i: (0, 0))
    return pl.pallas_call(
        _dense_body,
        grid=(_N // _BLK,),
        in_specs=[row_spec, row_spec, den_spec, den_spec, row_spec,
                  pl.BlockSpec((_D, _D), lambda i: (0, 0)),
                  vec_spec, vec_spec, vec_spec],
        out_specs=row_spec,
        out_shape=jax.ShapeDtypeStruct((_N, _D), jnp.float32),
    )(num0, num1, den0, den1, x, w, b, g, be)


def _per_worker_pad(arr, dtype):
    a = arr.reshape(_NW, _EPW)
    padded = jnp.concatenate(
        [a, jnp.zeros((_NW, _EPT - _EPW), dtype)], axis=1)
    return padded.reshape(_EPAD)


def kernel(node_attr, edge_index, batch_idx, adv_atts, W0, b0, g0, be0,
           W1, b1, g1, be1):
    src = _per_worker_pad(edge_index[0], jnp.int32)
    dst = _per_worker_pad(edge_index[1], jnp.int32)
    att0 = _per_worker_pad(adv_atts[0], jnp.float32)
    att1 = _per_worker_pad(adv_atts[1], jnp.float32)

    x = node_attr
    for att, w, b, g, be in ((att0, W0, b0, g0, be0),
                             (att1, W1, b1, g1, be1)):
        u, sden = _sc_edge_pass(x, src, dst, att)
        x = _dense_layer(u[0, :_N], u[1, :_N],
                         sden[0, :_N].reshape(_N, 1),
                         sden[1, :_N].reshape(_N, 1),
                         x, w, b.reshape(1, _D), g.reshape(1, _D),
                         be.reshape(1, _D))
    return x


# speedup vs baseline: 2.2715x; 1.0895x over previous
import functools

import jax
import jax.numpy as jnp
from jax import lax
from jax.experimental import pallas as pl
from jax.experimental.pallas import tpu as pltpu
from jax.experimental.pallas import tpu_sc as plsc

_N = 10000
_D = 128
_E = 320000
_BLK = 1000

_NCORES = 2
_NSUB = 16
_NW = _NCORES * _NSUB
_CH = 128
_NP = 10240
_RPT = _NP // _NSUB
_NCH0 = 65
_NCH1 = 92
_NCHMAX = 92
_EPT = _NCHMAX * _CH
_EPW0 = 8320
_EPW1 = 11680
_EPAD = _NW * _EPT


def _sc_edge_body(x_hbm, src_hbm, dst_hbm, att_hbm, u_out, s_out,
                  idxs_v, idxd_v, att_v, rows_v, zrow_v, zs_v, u_sh, s_sh,
                  sem):
    c = lax.axis_index("c")
    s = lax.axis_index("s")
    w = c * _NSUB + s
    zv = jnp.zeros((16,), jnp.float32)

    def zrow_body(i, carry):
        for j in range(8):
            zrow_v[i, pl.ds(j * 16, 16)] = zv
        return carry
    lax.fori_loop(0, _CH, zrow_body, 0)

    def zs_body(i, carry):
        zs_v[pl.ds(i * 16, 16)] = zv
        return carry
    lax.fori_loop(0, _RPT // 16, zs_body, 0)

    row0 = s * _RPT
    for t in range(_RPT // _CH):
        pltpu.sync_copy(zrow_v, u_sh.at[pl.ds(row0 + t * _CH, _CH)])
    pltpu.sync_copy(zs_v, s_sh.at[pl.ds(row0, _RPT)])
    plsc.subcore_barrier()

    base = w * _EPT
    nch = _NCH0 + (_NCH1 - _NCH0) * c

    def chunk_body(i, carry):
        @pl.when(i < nch)
        def _():
            off = base + i * _CH
            pltpu.sync_copy(src_hbm.at[pl.ds(off, _CH)], idxs_v)
            pltpu.sync_copy(dst_hbm.at[pl.ds(off, _CH)], idxd_v)
            pltpu.sync_copy(att_hbm.at[pl.ds(off, _CH)], att_v)
            pltpu.async_copy(x_hbm.at[idxs_v], rows_v, sem).wait()

            def scale_body(g, carry2):
                av = att_v[pl.ds(g * 16, 16)]
                for l in range(16):
                    a = av[l]
                    k = g * 16 + l
                    for j in range(8):
                        sl = pl.ds(j * 16, 16)
                        rows_v[k, sl] = rows_v[k, sl] * a
                return carry2
            lax.fori_loop(0, _CH // 16, scale_body, 0)

            pltpu.sync_copy(rows_v, u_sh.at[idxd_v], add=True)
            pltpu.sync_copy(att_v, s_sh.at[idxd_v], add=True)
        return carry
    lax.fori_loop(0, _NCHMAX, chunk_body, 0)
    plsc.subcore_barrier()

    pltpu.sync_copy(u_sh.at[pl.ds(row0, _RPT)],
                    u_out.at[c, pl.ds(row0, _RPT)])
    pltpu.sync_copy(s_sh.at[pl.ds(row0, _RPT)],
                    s_out.at[c, pl.ds(row0, _RPT)])


def _sc_edge_pass(x, src, dst, att):
    mesh = plsc.VectorSubcoreMesh(core_axis_name="c", subcore_axis_name="s")
    fn = functools.partial(
        pl.kernel,
        mesh=mesh,
        out_type=[
            jax.ShapeDtypeStruct((_NCORES, _NP, _D), jnp.float32),
            jax.ShapeDtypeStruct((_NCORES, _NP), jnp.float32),
        ],
        scratch_types=[
            pltpu.VMEM((_CH,), jnp.int32),
            pltpu.VMEM((_CH,), jnp.int32),
            pltpu.VMEM((_CH,), jnp.float32),
            pltpu.VMEM((_CH, _D), jnp.float32),
            pltpu.VMEM((_CH, _D), jnp.float32),
            pltpu.VMEM((_RPT,), jnp.float32),
            pltpu.VMEM_SHARED((_NP, _D), jnp.float32),
            pltpu.VMEM_SHARED((_NP,), jnp.float32),
            pltpu.SemaphoreType.DMA,
        ],
    )(_sc_edge_body)
    return fn(x, src, dst, att)


def _dense_body(num0_ref, num1_ref, den0_ref, den1_ref, x_ref, w_ref,
                b_ref, g_ref, be_ref, o_ref):
    num = num0_ref[...] + num1_ref[...]
    den = den0_ref[...] + den1_ref[...]
    x = x_ref[...]
    aggr = jnp.where(den > 0.0, num / jnp.where(den > 0.0, den, 1.0), 0.0)
    gelu = 0.5 * aggr * (1.0 + jax.lax.erf(aggr * 0.7071067811865476))
    h = gelu + x
    t = jax.lax.dot_general(h, w_ref[...], (((1,), (1,)), ((), ())),
                            preferred_element_type=jnp.float32)
    t = t + b_ref[...]
    mu = jnp.mean(t, axis=-1, keepdims=True)
    var = jnp.mean((t - mu) ** 2, axis=-1, keepdims=True)
    o_ref[...] = (t - mu) * jax.lax.rsqrt(var + 1e-5) * g_ref[...] + be_ref[...]


def _dense_layer(num0, num1, den0, den1, x, w, b, g, be):
    row_spec = pl.BlockSpec((_BLK, _D), lambda i: (i, 0))
    den_spec = pl.BlockSpec((_BLK, 1), lambda i: (i, 0))
    vec_spec = pl.BlockSpec((1, _D), lambda i: (0, 0))
    return pl.pallas_call(
        _dense_body,
        grid=(_N // _BLK,),
        in_specs=[row_spec, row_spec, den_spec, den_spec, row_spec,
                  pl.BlockSpec((_D, _D), lambda i: (0, 0)),
                  vec_spec, vec_spec, vec_spec],
        out_specs=row_spec,
        out_shape=jax.ShapeDtypeStruct((_N, _D), jnp.float32),
    )(num0, num1, den0, den1, x, w, b, g, be)


def _split_regions(arr, lens, starts):
    pos = starts[:, None] + jnp.arange(_EPT)[None, :]
    mask = jnp.arange(_EPT)[None, :] < lens[:, None]
    gath = jnp.clip(pos, 0, _E - 1)
    return jnp.where(mask, arr[gath], 0).reshape(_EPAD)


def kernel(node_attr, edge_index, batch_idx, adv_atts, W0, b0, g0, be0,
           W1, b1, g1, be1):
    lens = jnp.where(jnp.arange(_NW) < _NSUB, _EPW0, _EPW1)
    starts = jnp.concatenate(
        [jnp.zeros((1,), jnp.int32), jnp.cumsum(lens)[:-1]])
    src = _split_regions(edge_index[0], lens, starts)
    dst = _split_regions(edge_index[1], lens, starts)
    att0 = _split_regions(adv_atts[0].astype(jnp.float32), lens, starts)
    att1 = _split_regions(adv_atts[1].astype(jnp.float32), lens, starts)

    x = node_attr
    for att, w, b, g, be in ((att0, W0, b0, g0, be0),
                             (att1, W1, b1, g1, be1)):
        u, sden = _sc_edge_pass(x, src, dst, att)
        x = _dense_layer(u[0, :_N], u[1, :_N],
                         sden[0, :_N].reshape(_N, 1),
                         sden[1, :_N].reshape(_N, 1),
                         x, w, b.reshape(1, _D), g.reshape(1, _D),
                         be.reshape(1, _D))
    return x
